# baseline (speedup 1.0000x reference)
import functools

import jax
import jax.numpy as jnp
from jax import lax
from jax.experimental import pallas as pl
from jax.experimental.pallas import tpu as pltpu
from jax.experimental.pallas import tpu_sc as plsc

L = 16
NC = 2
NS = 16
NW = NC * NS
CHUNK = 128
TILE = 128


def _detile_body(u3_hbm, i3_hbm, utail_hbm, itail_hbm,
                 uflat_hbm, iflat_hbm, vbuf0, vbuf1, vbuf2, vbuf3,
                 sem_rd, sem_wr):
    V = u3_hbm.shape[2]
    nfull = V // TILE
    tail = V - nfull * TILE
    npad = (nfull + (1 if tail else 0)) * TILE
    cpw = (nfull + NW - 1) // NW
    wid = lax.axis_index("s") * NC + lax.axis_index("c")
    start = jnp.minimum(wid * cpw, nfull - cpw)
    off = pl.multiple_of(start * TILE, TILE)

    q = cpw // 4
    qs = [q, q, q, cpw - 3 * q]
    halves = []
    acc = 0
    for s in qs:
        halves.append((acc * TILE, s * TILE))
        acc += s
    jobs = []
    for src3, dstf in ((u3_hbm, uflat_hbm), (i3_hbm, iflat_hbm)):
        for R in range(2):
            for i in range(8):
                d = R * 8 + i
                for ho, hs in halves:
                    jobs.append((
                        src3.at[R, i, pl.ds(off + ho, hs)],
                        dstf.at[pl.ds(d * npad + off + ho, hs)]))

    bufs = [vbuf0, vbuf1, vbuf2, vbuf3]
    nbuf = len(bufs)
    depth = 2
    rds = [None] * nbuf
    wrs = [None] * nbuf
    sizes = [hs for _, hs in halves] * (len(jobs) // len(halves))
    for j in range(len(jobs) + depth):
        if j < len(jobs):
            b = j % nbuf
            if wrs[b] is not None:
                wrs[b].wait()
                wrs[b] = None
            rds[b] = pltpu.async_copy(
                jobs[j][0], bufs[b].at[pl.ds(0, sizes[j])], sem_rd)
        if j >= depth:
            k = j - depth
            b2 = k % nbuf
            rds[b2].wait()
            wrs[b2] = pltpu.async_copy(
                bufs[b2].at[pl.ds(0, sizes[k])], jobs[k][1], sem_wr)
    for w in wrs:
        if w is not None:
            w.wait()

    if tail:
        @pl.when(wid == 0)
        def _():
            pltpu.async_copy(
                utail_hbm, vbuf0.at[pl.ds(0, tail * L)], sem_rd).wait()
            pltpu.async_copy(
                itail_hbm, vbuf1.at[pl.ds(0, tail * L)], sem_rd).wait()
            cps2 = []
            for d in range(L):
                cps2.append(pltpu.async_copy(
                    vbuf0.at[pl.ds(d * tail, tail)],
                    uflat_hbm.at[pl.ds(d * npad + nfull * TILE, tail)],
                    sem_wr))
                cps2.append(pltpu.async_copy(
                    vbuf1.at[pl.ds(d * tail, tail)],
                    iflat_hbm.at[pl.ds(d * npad + nfull * TILE, tail)],
                    sem_wr))
            for cp in cps2:
                cp.wait()


def _mf_body(uidx_hbm, iidx_hbm, usr_nid_hbm, itm_nid_hbm,
             u1_hbm, i1_hbm, out_hbm,
             uidx_v, iidx_v, cu_v, ci_v, fu_v, fi_v,
             urows_v, irows_v, out_v,
             sem_idx, sem_rows):
    npad = u1_hbm.shape[0] // L
    wid = lax.axis_index("s") * NC + lax.axis_index("c")
    nchunk = uidx_v.shape[0]
    epw = nchunk * CHUNK
    base_row = wid * nchunk

    pltpu.sync_copy(uidx_hbm.at[pl.ds(base_row, nchunk)], uidx_v)
    pltpu.sync_copy(iidx_hbm.at[pl.ds(base_row, nchunk)], iidx_v)

    cps = []
    for c in range(nchunk):
        cps.append(pltpu.async_copy(
            usr_nid_hbm.at[uidx_v.at[c]], cu_v.at[c], sem_idx))
        cps.append(pltpu.async_copy(
            itm_nid_hbm.at[iidx_v.at[c]], ci_v.at[c], sem_idx))
    for cp in cps:
        cp.wait()

    spg = CHUNK // L

    def build(s, carry):
        c = s // spg
        o = (s % spg) * L
        rowu = cu_v[c, pl.ds(o, L)]
        rowi = ci_v[c, pl.ds(o, L)]
        for d in range(L):
            dv = jnp.full((L,), d * npad, jnp.int32)
            fu_v[d * nchunk + c, pl.ds(o, L)] = rowu + dv
            fi_v[d * nchunk + c, pl.ds(o, L)] = rowi + dv
        return carry

    lax.fori_loop(0, epw // L, build, 0)

    cps = []
    for d in range(L):
        for c in range(nchunk):
            cps.append(pltpu.async_copy(
                u1_hbm.at[fu_v.at[d * nchunk + c]],
                urows_v.at[d, pl.ds(c * CHUNK, CHUNK)], sem_rows))
            cps.append(pltpu.async_copy(
                i1_hbm.at[fi_v.at[d * nchunk + c]],
                irows_v.at[d, pl.ds(c * CHUNK, CHUNK)], sem_rows))
    for cp in cps:
        cp.wait()

    def group(g, carry):
        b = g * L
        acc = urows_v[0, pl.ds(b, L)] * irows_v[0, pl.ds(b, L)]
        for d in range(1, L):
            acc = acc + urows_v[d, pl.ds(b, L)] * irows_v[d, pl.ds(b, L)]
        out_v[pl.ds(b, L)] = acc
        return carry

    lax.fori_loop(0, epw // L, group, 0)

    pltpu.sync_copy(out_v, out_hbm.at[pl.ds(wid * epw, epw)])


def kernel(usr_n_id, itm_n_id, edge_label_index, usr_table, itm_table):
    B = usr_n_id.shape[0]
    epw = B // NW
    nchunk = epw // CHUNK

    usr_idx = edge_label_index[0].astype(jnp.int32).reshape(B // CHUNK, CHUNK)
    itm_idx = edge_label_index[1].astype(jnp.int32).reshape(B // CHUNK, CHUNK)
    usr_n_id = usr_n_id.astype(jnp.int32)
    itm_n_id = itm_n_id.astype(jnp.int32)

    V = usr_table.shape[0]
    npad = ((V + TILE - 1) // TILE) * TILE
    u3 = usr_table.T.reshape(2, 8, V)
    i3 = itm_table.T.reshape(2, 8, V)
    nfull = V // TILE
    utail = usr_table[nfull * TILE:, :].T.reshape(-1)
    itail = itm_table[nfull * TILE:, :].T.reshape(-1)

    cpw = (nfull + NW - 1) // NW
    hbuf = (cpw - 3 * (cpw // 4)) * TILE

    mesh = plsc.VectorSubcoreMesh(core_axis_name="c", subcore_axis_name="s")

    detile = functools.partial(
        pl.kernel,
        mesh=mesh,
        compiler_params=pltpu.CompilerParams(use_tc_tiling_on_sc=True),
        out_type=(jax.ShapeDtypeStruct((L * npad,), jnp.float32),
                  jax.ShapeDtypeStruct((L * npad,), jnp.float32)),
        scratch_types=[
            pltpu.VMEM((hbuf,), jnp.float32),
            pltpu.VMEM((hbuf,), jnp.float32),
            pltpu.VMEM((hbuf,), jnp.float32),
            pltpu.VMEM((hbuf,), jnp.float32),
            pltpu.SemaphoreType.DMA,
            pltpu.SemaphoreType.DMA,
        ],
    )(_detile_body)
    u1, i1 = detile(u3, i3, utail, itail)

    score = functools.partial(
        pl.kernel,
        mesh=mesh,
        compiler_params=pltpu.CompilerParams(use_tc_tiling_on_sc=False),
        out_type=jax.ShapeDtypeStruct((B,), jnp.float32),
        scratch_types=[
            pltpu.VMEM((nchunk, CHUNK), jnp.int32),
            pltpu.VMEM((nchunk, CHUNK), jnp.int32),
            pltpu.VMEM((nchunk, CHUNK), jnp.int32),
            pltpu.VMEM((nchunk, CHUNK), jnp.int32),
            pltpu.VMEM((L * nchunk, CHUNK), jnp.int32),
            pltpu.VMEM((L * nchunk, CHUNK), jnp.int32),
            pltpu.VMEM((L, epw), jnp.float32),
            pltpu.VMEM((L, epw), jnp.float32),
            pltpu.VMEM((epw,), jnp.float32),
            pltpu.SemaphoreType.DMA,
            pltpu.SemaphoreType.DMA,
        ],
    )(_mf_body)
    return score(usr_idx, itm_idx, usr_n_id, itm_n_id, u1, i1)


# speedup vs baseline: 1.0369x; 1.0369x over previous
import functools

import jax
import jax.numpy as jnp
from jax import lax
from jax.experimental import pallas as pl
from jax.experimental.pallas import tpu as pltpu
from jax.experimental.pallas import tpu_sc as plsc

L = 16
NC = 2
NS = 16
NW = NC * NS
CHUNK = 128
TILE = 128


def _detile_body(u3_hbm, i3_hbm, utail_hbm, itail_hbm,
                 uflat_hbm, iflat_hbm, vbuf0, vbuf1, vbuf2, vbuf3,
                 sem_rd, sem_wr):
    V = u3_hbm.shape[2]
    nfull = V // TILE
    tail = V - nfull * TILE
    npad = (nfull + (1 if tail else 0)) * TILE
    cpw = (nfull + NW - 1) // NW
    wid = lax.axis_index("s") * NC + lax.axis_index("c")
    start = jnp.minimum(wid * cpw, nfull - cpw)
    off = pl.multiple_of(start * TILE, TILE)

    halves = [(0, (cpw // 2) * TILE),
              ((cpw // 2) * TILE, (cpw - cpw // 2) * TILE)]
    jobs = []
    for src3, dstf in ((u3_hbm, uflat_hbm), (i3_hbm, iflat_hbm)):
        for R in range(2):
            for i in range(8):
                d = R * 8 + i
                for ho, hs in halves:
                    jobs.append((
                        src3.at[R, i, pl.ds(off + ho, hs)],
                        dstf.at[pl.ds(d * npad + off + ho, hs)]))

    bufs = [vbuf0, vbuf1, vbuf2, vbuf3]
    nbuf = len(bufs)
    depth = 3
    rds = [None] * nbuf
    wrs = [None] * nbuf
    sizes = [hs for _, hs in halves] * (len(jobs) // 2)
    for j in range(len(jobs) + depth):
        if j < len(jobs):
            b = j % nbuf
            if wrs[b] is not None:
                wrs[b].wait()
                wrs[b] = None
            rds[b] = pltpu.async_copy(
                jobs[j][0], bufs[b].at[pl.ds(0, sizes[j])], sem_rd)
        if j >= depth:
            k = j - depth
            b2 = k % nbuf
            rds[b2].wait()
            wrs[b2] = pltpu.async_copy(
                bufs[b2].at[pl.ds(0, sizes[k])], jobs[k][1], sem_wr)
    for w in wrs:
        if w is not None:
            w.wait()

    if tail:
        @pl.when(wid == 0)
        def _():
            pltpu.async_copy(
                utail_hbm, vbuf0.at[pl.ds(0, tail * L)], sem_rd).wait()
            pltpu.async_copy(
                itail_hbm, vbuf1.at[pl.ds(0, tail * L)], sem_rd).wait()
            cps2 = []
            for d in range(L):
                cps2.append(pltpu.async_copy(
                    vbuf0.at[pl.ds(d * tail, tail)],
                    uflat_hbm.at[pl.ds(d * npad + nfull * TILE, tail)],
                    sem_wr))
                cps2.append(pltpu.async_copy(
                    vbuf1.at[pl.ds(d * tail, tail)],
                    iflat_hbm.at[pl.ds(d * npad + nfull * TILE, tail)],
                    sem_wr))
            for cp in cps2:
                cp.wait()


def _mf_body(uidx_hbm, iidx_hbm, usr_nid_hbm, itm_nid_hbm,
             u1_hbm, i1_hbm, out_hbm,
             uidx_v, iidx_v, cu_v, ci_v, fu_v, fi_v,
             urows_v, irows_v, out_v,
             sem_idx, sem_rows):
    npad = u1_hbm.shape[0] // L
    wid = lax.axis_index("s") * NC + lax.axis_index("c")
    nchunk = uidx_v.shape[0]
    epw = nchunk * CHUNK
    base_row = wid * nchunk

    pltpu.sync_copy(uidx_hbm.at[pl.ds(base_row, nchunk)], uidx_v)
    pltpu.sync_copy(iidx_hbm.at[pl.ds(base_row, nchunk)], iidx_v)

    cps = []
    for c in range(nchunk):
        cps.append(pltpu.async_copy(
            usr_nid_hbm.at[uidx_v.at[c]], cu_v.at[c], sem_idx))
        cps.append(pltpu.async_copy(
            itm_nid_hbm.at[iidx_v.at[c]], ci_v.at[c], sem_idx))
    for cp in cps:
        cp.wait()

    spg = CHUNK // L

    def build(s, carry):
        c = s // spg
        o = (s % spg) * L
        rowu = cu_v[c, pl.ds(o, L)]
        rowi = ci_v[c, pl.ds(o, L)]
        for d in range(L):
            dv = jnp.full((L,), d * npad, jnp.int32)
            fu_v[d * nchunk + c, pl.ds(o, L)] = rowu + dv
            fi_v[d * nchunk + c, pl.ds(o, L)] = rowi + dv
        return carry

    lax.fori_loop(0, epw // L, build, 0)

    cps = []
    for d in range(L):
        for c in range(nchunk):
            cps.append(pltpu.async_copy(
                u1_hbm.at[fu_v.at[d * nchunk + c]],
                urows_v.at[d, pl.ds(c * CHUNK, CHUNK)], sem_rows))
            cps.append(pltpu.async_copy(
                i1_hbm.at[fi_v.at[d * nchunk + c]],
                irows_v.at[d, pl.ds(c * CHUNK, CHUNK)], sem_rows))
    for cp in cps:
        cp.wait()

    def group(g, carry):
        b = g * L
        acc = urows_v[0, pl.ds(b, L)] * irows_v[0, pl.ds(b, L)]
        for d in range(1, L):
            acc = acc + urows_v[d, pl.ds(b, L)] * irows_v[d, pl.ds(b, L)]
        out_v[pl.ds(b, L)] = acc
        return carry

    lax.fori_loop(0, epw // L, group, 0)

    pltpu.sync_copy(out_v, out_hbm.at[pl.ds(wid * epw, epw)])


def kernel(usr_n_id, itm_n_id, edge_label_index, usr_table, itm_table):
    B = usr_n_id.shape[0]
    epw = B // NW
    nchunk = epw // CHUNK

    usr_idx = edge_label_index[0].astype(jnp.int32).reshape(B // CHUNK, CHUNK)
    itm_idx = edge_label_index[1].astype(jnp.int32).reshape(B // CHUNK, CHUNK)
    usr_n_id = usr_n_id.astype(jnp.int32)
    itm_n_id = itm_n_id.astype(jnp.int32)

    V = usr_table.shape[0]
    npad = ((V + TILE - 1) // TILE) * TILE
    u3 = usr_table.T.reshape(2, 8, V)
    i3 = itm_table.T.reshape(2, 8, V)
    nfull = V // TILE
    utail = usr_table[nfull * TILE:, :].T.reshape(-1)
    itail = itm_table[nfull * TILE:, :].T.reshape(-1)

    cpw = (nfull + NW - 1) // NW
    hbuf = (cpw - cpw // 2) * TILE

    mesh = plsc.VectorSubcoreMesh(core_axis_name="c", subcore_axis_name="s")

    detile = functools.partial(
        pl.kernel,
        mesh=mesh,
        compiler_params=pltpu.CompilerParams(use_tc_tiling_on_sc=True),
        out_type=(jax.ShapeDtypeStruct((L * npad,), jnp.float32),
                  jax.ShapeDtypeStruct((L * npad,), jnp.float32)),
        scratch_types=[
            pltpu.VMEM((hbuf,), jnp.float32),
            pltpu.VMEM((hbuf,), jnp.float32),
            pltpu.VMEM((hbuf,), jnp.float32),
            pltpu.VMEM((hbuf,), jnp.float32),
            pltpu.SemaphoreType.DMA,
            pltpu.SemaphoreType.DMA,
        ],
    )(_detile_body)
    u1, i1 = detile(u3, i3, utail, itail)

    score = functools.partial(
        pl.kernel,
        mesh=mesh,
        compiler_params=pltpu.CompilerParams(use_tc_tiling_on_sc=False),
        out_type=jax.ShapeDtypeStruct((B,), jnp.float32),
        scratch_types=[
            pltpu.VMEM((nchunk, CHUNK), jnp.int32),
            pltpu.VMEM((nchunk, CHUNK), jnp.int32),
            pltpu.VMEM((nchunk, CHUNK), jnp.int32),
            pltpu.VMEM((nchunk, CHUNK), jnp.int32),
            pltpu.VMEM((L * nchunk, CHUNK), jnp.int32),
            pltpu.VMEM((L * nchunk, CHUNK), jnp.int32),
            pltpu.VMEM((L, epw), jnp.float32),
            pltpu.VMEM((L, epw), jnp.float32),
            pltpu.VMEM((epw,), jnp.float32),
            pltpu.SemaphoreType.DMA,
            pltpu.SemaphoreType.DMA,
        ],
    )(_mf_body)
    return score(usr_idx, itm_idx, usr_n_id, itm_n_id, u1, i1)
